# baseline (speedup 1.0000x reference)
import jax
import jax.numpy as jnp
from jax import lax
from jax.experimental import pallas as pl
from jax.experimental.pallas import tpu as pltpu
from jax.experimental.pallas import tpu_sc as plsc

N = 8192
NV = N // 16
NLEV = 9
ROWS_PER_W = 4
NW = 32
_UNROLL = 4


def _sort16(x, desc):
  if desc:
    return plsc.sort_key_val(x, x, descending=True)[0]
  return jnp.sort(x)


def _cmp2(a, b, desc):
  lo = jnp.minimum(a, b)
  hi = jnp.maximum(a, b)
  return (hi, lo) if desc else (lo, hi)


def _body(u_hbm, v_hbm, out_hbm, buf, res_v, sem):
  c = lax.axis_index("c")
  s = lax.axis_index("s")
  wid = c * 16 + s
  lanes = lax.iota(jnp.int32, 16)

  row0 = wid * ROWS_PER_W
  pltpu.async_copy(u_hbm.at[row0], buf.at[pl.ds(0, N)], sem)
  pltpu.async_copy(v_hbm.at[row0], buf.at[pl.ds(N, N)], sem)

  def row_body(r, res):
    row = wid * ROWS_PER_W + r
    half = r & 1
    hbase = half * 2 * N
    voff = half * 2 * NV

    def vreg(i):
      return pl.ds((voff + i) * 16, 16)

    pltpu.make_async_copy(
        u_hbm.at[row], buf.at[pl.ds(hbase, N)], sem).wait()
    pltpu.make_async_copy(
        v_hbm.at[row], buf.at[pl.ds(hbase + N, N)], sem).wait()

    @pl.when(r < ROWS_PER_W - 1)
    def _prefetch():
      obase = (1 - half) * 2 * N
      pltpu.async_copy(u_hbm.at[row + 1], buf.at[pl.ds(obase, N)], sem)
      pltpu.async_copy(v_hbm.at[row + 1], buf.at[pl.ds(obase + N, N)], sem)

    def network(xs, desc, final_vsort):
      xs = list(xs)
      L = len(xs).bit_length() - 1
      for si in range(L):
        st = 1 << (L - 1 - si)
        for t in range(len(xs)):
          if (t // st) % 2 == 0:
            xs[t], xs[t + st] = _cmp2(xs[t], xs[t + st], desc)
      if final_vsort:
        xs = [_sort16(x, desc) for x in xs]
      return xs

    def level1(par):
      @plsc.parallel_loop(0, NV // 2, unroll=_UNROLL)
      def _l1(q):
        vi = 4 * q + 2 * par
        s0 = _sort16(buf[vreg(vi)], False)
        s1 = _sort16(buf[vreg(vi + 1)], True)
        lo, hi = _cmp2(s0, s1, par == 1)
        buf[vreg(vi)] = _sort16(lo, par == 1)
        buf[vreg(vi + 1)] = _sort16(hi, par == 1)

    level1(0)
    level1(1)

    def a_chunk(j, La, vsort_end):
      K = 1 << (j - 1)
      H = 1 << (La - 1)
      Qa = K // H
      unroll = {2: _UNROLL, 3: 2, 4: 1}[La]

      def make(par):
        split = j < NLEV
        trips = NV // H // 2 if split else NV // H

        @plsc.parallel_loop(0, trips, unroll=unroll)
        def _a(g):
          if split:
            m = 2 * (g // Qa) + par
          else:
            m = g // Qa
          i = g & (Qa - 1)
          desc = split and par == 1
          base = m * 2 * K
          l = []
          h = []
          for t in range(H):
            x = buf[vreg(base + i + t * Qa)]
            y = buf[vreg(base + K + i + t * Qa)]
            lo, hi = _cmp2(x, y, desc)
            l.append(lo)
            h.append(hi)
          l = network(l, desc, vsort_end)
          h = network(h, desc, vsort_end)
          for t in range(H):
            buf[vreg(base + i + t * Qa)] = l[t]
            buf[vreg(base + K + i + t * Qa)] = h[t]

      if j < NLEV:
        make(0)
        make(1)
      else:
        make(0)

    def chunk_pass(j, dists, vsort, acc_mode):
      L = len(dists)
      G = 1 << L
      b = dists[-1].bit_length() - 1
      unroll = {2: _UNROLL, 4: _UNROLL, 8: 2, 16: 1}[G]

      def group_base(g):
        return ((g >> b) << (b + L)) | (g & ((1 << b) - 1))

      if acc_mode:

        @plsc.parallel_loop(
            0, NV // G, unroll=unroll, carry=jnp.zeros((16,), jnp.float32))
        def _chunk_acc(g, acc):
          v0 = group_base(g)
          xu = network(
              [buf[vreg(v0 + (t << b))] for t in range(G)], False, True)
          xv = network(
              [buf[vreg(NV + v0 + (t << b))] for t in range(G)], False, True)
          for a_u, a_v in zip(xu, xv):
            acc = acc + jnp.abs(a_u - a_v)
          return acc

        return _chunk_acc

      def make(par):
        split = j < NLEV
        sbit = j - L
        trips = 2 * NV // G // 2 if split else 2 * NV // G

        @plsc.parallel_loop(0, trips, unroll=unroll)
        def _chunk(g):
          if split:
            gg = ((g >> sbit) << (sbit + 1)) | (g & ((1 << sbit) - 1))
            gg = gg | (par << sbit)
          else:
            gg = g
          desc = split and par == 1
          v0 = group_base(gg)
          xs = network(
              [buf[vreg(v0 + (t << b))] for t in range(G)], desc, vsort)
          for t in range(G):
            buf[vreg(v0 + (t << b))] = xs[t]

      if j < NLEV:
        make(0)
        make(1)
      else:
        make(0)

    plan = {
        5: (3, [[2, 1]]),
        6: (3, [[4, 2, 1]]),
        7: (4, [[4, 2, 1]]),
        8: (4, [[8, 4, 2, 1]]),
        9: (3, [[32, 16, 8], [4, 2, 1]]),
    }
    acc_final = [None]
    for j in range(2, NLEV + 1):
      if j <= 4:
        a_chunk(j, j, True)
        continue
      La, rem = plan[j]
      a_chunk(j, La, False)
      for ci, dists in enumerate(rem):
        last = ci == len(rem) - 1
        if last and j == NLEV:
          acc_final[0] = chunk_pass(j, dists, True, True)
        else:
          chunk_pass(j, dists, last, False)

    total = jnp.sum(acc_final[0]) * (1.0 / N)
    return jnp.where(lanes == r, total, res)

  res = lax.fori_loop(0, ROWS_PER_W, row_body, jnp.zeros((16,), jnp.float32))
  res_v[...] = res
  pltpu.sync_copy(res_v, out_hbm.at[wid])


def kernel(u_values, v_values):
  mesh = plsc.VectorSubcoreMesh(core_axis_name="c", subcore_axis_name="s")
  out = pl.kernel(
      _body,
      out_type=jax.ShapeDtypeStruct((NW, 16), jnp.float32),
      mesh=mesh,
      compiler_params=pltpu.CompilerParams(needs_layout_passes=False),
      scratch_types=[
          pltpu.VMEM((4 * N,), jnp.float32),
          pltpu.VMEM((16,), jnp.float32),
          pltpu.SemaphoreType.DMA,
      ],
  )(u_values, v_values)
  return out[:, :ROWS_PER_W].reshape(128)


# speedup vs baseline: 1.0390x; 1.0390x over previous
import jax
import jax.numpy as jnp
from jax import lax
from jax.experimental import pallas as pl
from jax.experimental.pallas import tpu as pltpu
from jax.experimental.pallas import tpu_sc as plsc

N = 8192
NV = N // 16
NLEV = 9
ROWS_PER_W = 4
NW = 32
_UNROLL = 4


def _body(u_hbm, v_hbm, out_hbm, buf, res_v, sem):
  c = lax.axis_index("c")
  s = lax.axis_index("s")
  wid = c * 16 + s
  lanes = lax.iota(jnp.int32, 16)

  row0 = wid * ROWS_PER_W
  pltpu.async_copy(u_hbm.at[row0], buf.at[pl.ds(0, N)], sem)
  pltpu.async_copy(v_hbm.at[row0], buf.at[pl.ds(N, N)], sem)

  def row_body(r, res):
    row = wid * ROWS_PER_W + r
    half = r & 1
    hbase = half * 2 * N
    voff = half * 2 * NV

    def vreg(i):
      return pl.ds((voff + i) * 16, 16)

    pltpu.make_async_copy(
        u_hbm.at[row], buf.at[pl.ds(hbase, N)], sem).wait()
    pltpu.make_async_copy(
        v_hbm.at[row], buf.at[pl.ds(hbase + N, N)], sem).wait()

    @pl.when(r < ROWS_PER_W - 1)
    def _prefetch():
      obase = (1 - half) * 2 * N
      pltpu.async_copy(u_hbm.at[row + 1], buf.at[pl.ds(obase, N)], sem)
      pltpu.async_copy(v_hbm.at[row + 1], buf.at[pl.ds(obase + N, N)], sem)

    @plsc.parallel_loop(0, NV, unroll=_UNROLL)
    def _level1(p):
      vi = 2 * p
      x = jnp.sort(buf[vreg(vi)])
      y = buf[vreg(vi + 1)]
      ry = plsc.sort_key_val(y, y, descending=True)[0]
      buf[vreg(vi)] = jnp.sort(jnp.minimum(x, ry))
      buf[vreg(vi + 1)] = jnp.sort(jnp.maximum(x, ry))

    def network(xs, dists, final_vsort):
      L = len(dists)
      xs = list(xs)
      for si in range(L):
        st = 1 << (L - 1 - si)
        for t in range(1 << L):
          if (t // st) % 2 == 0:
            a, b = xs[t], xs[t + st]
            xs[t] = jnp.minimum(a, b)
            xs[t + st] = jnp.maximum(a, b)
      if final_vsort:
        xs = [jnp.sort(x) for x in xs]
      return xs

    def a_chunk(j, La, vsort_end):
      K = 1 << (j - 1)
      H = 1 << (La - 1)
      Qa = K // H
      unroll = {2: _UNROLL, 3: 2, 4: 1}[La]

      @plsc.parallel_loop(0, NV // H, unroll=unroll)
      def _a(g):
        m = g // Qa
        i = g & (Qa - 1)
        base = m * 2 * K
        l = []
        h = []
        for t in range(H):
          x = buf[vreg(base + i + t * Qa)]
          ry = jnp.flip(buf[vreg(base + 2 * K - 1 - i - t * Qa)])
          l.append(jnp.minimum(x, ry))
          h.append(jnp.maximum(x, ry))
        w = h[::-1]
        l = network(l, [0] * (La - 1), vsort_end)
        w = network(w, [0] * (La - 1), vsort_end)
        for t in range(H):
          buf[vreg(base + i + t * Qa)] = l[t]
          buf[vreg(base + 2 * K - 1 - i - (H - 1 - t) * Qa)] = w[t]

    def chunk_pass(dists, vsort, acc_mode):
      L = len(dists)
      G = 1 << L
      b = dists[-1].bit_length() - 1
      unroll = {2: _UNROLL, 4: _UNROLL, 8: 2, 16: 1}[G]

      if acc_mode:

        @plsc.parallel_loop(
            0, NV // G, unroll=unroll, carry=jnp.zeros((16,), jnp.float32))
        def _chunk_acc(g, acc):
          v0 = ((g >> b) << (b + L)) | (g & ((1 << b) - 1))
          xu = network(
              [buf[vreg(v0 + (t << b))] for t in range(G)], dists, True)
          xv = network(
              [buf[vreg(NV + v0 + (t << b))] for t in range(G)], dists, True)
          for a_u, a_v in zip(xu, xv):
            acc = acc + jnp.abs(a_u - a_v)
          return acc

        return _chunk_acc

      @plsc.parallel_loop(0, 2 * NV // G, unroll=unroll)
      def _chunk(g):
        v0 = ((g >> b) << (b + L)) | (g & ((1 << b) - 1))
        xs = network(
            [buf[vreg(v0 + (t << b))] for t in range(G)], dists, vsort)
        for t in range(G):
          buf[vreg(v0 + (t << b))] = xs[t]

    plan = {
        5: (3, [[2, 1]]),
        6: (3, [[4, 2, 1]]),
        7: (4, [[4, 2, 1]]),
        8: (4, [[8, 4, 2, 1]]),
        9: (3, [[32, 16, 8], [4, 2, 1]]),
    }
    acc_final = [None]
    for j in range(2, NLEV + 1):
      if j <= 4:
        a_chunk(j, j, True)
        continue
      La, rem = plan[j]
      a_chunk(j, La, False)
      for ci, dists in enumerate(rem):
        last = ci == len(rem) - 1
        if last and j == NLEV:
          acc_final[0] = chunk_pass(dists, True, True)
        else:
          chunk_pass(dists, last, False)

    total = jnp.sum(acc_final[0]) * (1.0 / N)
    return jnp.where(lanes == r, total, res)

  res = lax.fori_loop(0, ROWS_PER_W, row_body, jnp.zeros((16,), jnp.float32))
  res_v[...] = res
  pltpu.sync_copy(res_v, out_hbm.at[wid])


def kernel(u_values, v_values):
  mesh = plsc.VectorSubcoreMesh(core_axis_name="c", subcore_axis_name="s")
  out = pl.kernel(
      _body,
      out_type=jax.ShapeDtypeStruct((NW, 16), jnp.float32),
      mesh=mesh,
      compiler_params=pltpu.CompilerParams(needs_layout_passes=False),
      scratch_types=[
          pltpu.VMEM((4 * N,), jnp.float32),
          pltpu.VMEM((16,), jnp.float32),
          pltpu.SemaphoreType.DMA,
      ],
  )(u_values, v_values)
  return out[:, :ROWS_PER_W].reshape(128)


# speedup vs baseline: 1.0404x; 1.0013x over previous
import jax
import jax.numpy as jnp
from jax import lax
from jax.experimental import pallas as pl
from jax.experimental.pallas import tpu as pltpu
from jax.experimental.pallas import tpu_sc as plsc

N = 8192
NV = N // 16
NLEV = 9
ROWS_PER_W = 4
NW = 32
_UNROLL = 4


def _body(u_hbm, v_hbm, out_hbm, buf, res_v, sem):
  c = lax.axis_index("c")
  s = lax.axis_index("s")
  wid = c * 16 + s
  lanes = lax.iota(jnp.int32, 16)

  row0 = wid * ROWS_PER_W
  pltpu.async_copy(u_hbm.at[row0], buf.at[pl.ds(0, N)], sem)
  pltpu.async_copy(v_hbm.at[row0], buf.at[pl.ds(N, N)], sem)

  def row_body(r, res):
    row = wid * ROWS_PER_W + r
    half = r & 1
    hbase = half * 2 * N
    voff = half * 2 * NV

    def vreg(i):
      return pl.ds((voff + i) * 16, 16)

    pltpu.make_async_copy(
        u_hbm.at[row], buf.at[pl.ds(hbase, N)], sem).wait()
    pltpu.make_async_copy(
        v_hbm.at[row], buf.at[pl.ds(hbase + N, N)], sem).wait()

    @pl.when(r < ROWS_PER_W - 1)
    def _prefetch():
      obase = (1 - half) * 2 * N
      pltpu.async_copy(u_hbm.at[row + 1], buf.at[pl.ds(obase, N)], sem)
      pltpu.async_copy(v_hbm.at[row + 1], buf.at[pl.ds(obase + N, N)], sem)

    @plsc.parallel_loop(0, NV, unroll=_UNROLL)
    def _level1(p):
      vi = 2 * p
      x = jnp.sort(buf[vreg(vi)])
      y = buf[vreg(vi + 1)]
      ry = plsc.sort_key_val(y, y, descending=True)[0]
      buf[vreg(vi)] = jnp.sort(jnp.minimum(x, ry))
      buf[vreg(vi + 1)] = jnp.sort(jnp.maximum(x, ry))

    def network(xs, dists, final_vsort):
      L = len(dists)
      xs = list(xs)
      for si in range(L):
        st = 1 << (L - 1 - si)
        for t in range(1 << L):
          if (t // st) % 2 == 0:
            a, b = xs[t], xs[t + st]
            xs[t] = jnp.minimum(a, b)
            xs[t + st] = jnp.maximum(a, b)
      if final_vsort:
        xs = [jnp.sort(x) for x in xs]
      return xs

    def a_chunk(j, La, vsort_end):
      K = 1 << (j - 1)
      H = 1 << (La - 1)
      Qa = K // H
      unroll = {2: _UNROLL, 3: 2, 4: 1}[La]

      @plsc.parallel_loop(0, NV // H, unroll=unroll)
      def _a(g):
        m = g // Qa
        i = g & (Qa - 1)
        base = m * 2 * K
        l = []
        h = []
        for t in range(H):
          x = buf[vreg(base + i + t * Qa)]
          ry = jnp.flip(buf[vreg(base + 2 * K - 1 - i - t * Qa)])
          l.append(jnp.minimum(x, ry))
          h.append(jnp.maximum(x, ry))
        w = h[::-1]
        l = network(l, [0] * (La - 1), vsort_end)
        w = network(w, [0] * (La - 1), vsort_end)
        for t in range(H):
          buf[vreg(base + i + t * Qa)] = l[t]
          buf[vreg(base + 2 * K - 1 - i - (H - 1 - t) * Qa)] = w[t]

    def chunk_pass(dists, vsort, acc_mode):
      L = len(dists)
      G = 1 << L
      b = dists[-1].bit_length() - 1
      unroll = {2: _UNROLL, 4: _UNROLL, 8: 2, 16: 1}[G]

      if acc_mode:

        @plsc.parallel_loop(
            0, NV // G, unroll=unroll, carry=jnp.zeros((16,), jnp.float32))
        def _chunk_acc(g, acc):
          v0 = ((g >> b) << (b + L)) | (g & ((1 << b) - 1))
          xu = network(
              [buf[vreg(v0 + (t << b))] for t in range(G)], dists, True)
          xv = network(
              [buf[vreg(NV + v0 + (t << b))] for t in range(G)], dists, True)
          for a_u, a_v in zip(xu, xv):
            acc = acc + jnp.abs(a_u - a_v)
          return acc

        return _chunk_acc

      @plsc.parallel_loop(0, 2 * NV // G, unroll=unroll)
      def _chunk(g):
        v0 = ((g >> b) << (b + L)) | (g & ((1 << b) - 1))
        xs = network(
            [buf[vreg(v0 + (t << b))] for t in range(G)], dists, vsort)
        for t in range(G):
          buf[vreg(v0 + (t << b))] = xs[t]

    plan = {
        5: (3, [[2, 1]]),
        6: (3, [[4, 2, 1]]),
        7: (4, [[4, 2, 1]]),
        8: (4, [[8, 4, 2, 1]]),
        9: (4, [[16, 8, 4, 2], [1]]),
    }
    acc_final = [None]
    for j in range(2, NLEV + 1):
      if j <= 4:
        a_chunk(j, j, True)
        continue
      La, rem = plan[j]
      a_chunk(j, La, False)
      for ci, dists in enumerate(rem):
        last = ci == len(rem) - 1
        if last and j == NLEV:
          acc_final[0] = chunk_pass(dists, True, True)
        else:
          chunk_pass(dists, last, False)

    total = jnp.sum(acc_final[0]) * (1.0 / N)
    return jnp.where(lanes == r, total, res)

  res = lax.fori_loop(0, ROWS_PER_W, row_body, jnp.zeros((16,), jnp.float32))
  res_v[...] = res
  pltpu.sync_copy(res_v, out_hbm.at[wid])


def kernel(u_values, v_values):
  mesh = plsc.VectorSubcoreMesh(core_axis_name="c", subcore_axis_name="s")
  out = pl.kernel(
      _body,
      out_type=jax.ShapeDtypeStruct((NW, 16), jnp.float32),
      mesh=mesh,
      compiler_params=pltpu.CompilerParams(needs_layout_passes=False),
      scratch_types=[
          pltpu.VMEM((4 * N,), jnp.float32),
          pltpu.VMEM((16,), jnp.float32),
          pltpu.SemaphoreType.DMA,
      ],
  )(u_values, v_values)
  return out[:, :ROWS_PER_W].reshape(128)
